# Initial kernel scaffold; baseline (speedup 1.0000x reference)
import functools

import jax
import jax.numpy as jnp
from jax import lax
from jax.experimental import pallas as pl
from jax.experimental.pallas import tpu as pltpu
from jax.experimental.pallas import tpu_sc as plsc

N_U = 5000
N_I = 5000
N = 10000
D = 128
L = 3
E = 320000

NC = 2
NS = 16
CH = 128
EPT = 20096
NCHUNK = EPT // CH
E_PAD = EPT * NS
ROWS_PAD = 5008
RPT = ROWS_PAD // NS
PAD_ROW = ROWS_PAD - 1

_mesh = plsc.VectorSubcoreMesh(core_axis_name="c", subcore_axis_name="s")


@functools.partial(
    pl.kernel,
    out_type=jax.ShapeDtypeStruct((NC, ROWS_PAD, 16), jnp.float32),
    mesh=_mesh,
    scratch_types=[
        pltpu.VMEM((CH,), jnp.int32),
        pltpu.VMEM((CH, 16), jnp.float32),
        pltpu.VMEM((RPT, 16), jnp.float32),
        pltpu.VMEM_SHARED((ROWS_PAD, 16), jnp.float32),
    ],
)
def _deg_sc(rows_hbm, ones_hbm, zeros_hbm, out_hbm, ridx, ones_v, zb, acc):
    c = lax.axis_index("c")
    s = lax.axis_index("s")
    pltpu.sync_copy(zeros_hbm, zb)
    pltpu.sync_copy(zb, acc.at[pl.ds(s * RPT, RPT)])
    pltpu.sync_copy(ones_hbm, ones_v)
    plsc.subcore_barrier()

    def body(j, _):
        off = s * EPT + j * CH
        pltpu.sync_copy(rows_hbm.at[c, pl.ds(off, CH)], ridx)
        pltpu.sync_copy(ones_v, acc.at[ridx], add=True)
        return 0

    lax.fori_loop(0, NCHUNK, body, 0)
    plsc.subcore_barrier()
    pltpu.sync_copy(acc.at[pl.ds(s * RPT, RPT)], zb)
    pltpu.sync_copy(zb, out_hbm.at[c, pl.ds(s * RPT, RPT)])


@functools.partial(
    pl.kernel,
    out_type=jax.ShapeDtypeStruct((NC, ROWS_PAD, D), jnp.float32),
    mesh=_mesh,
    scratch_types=[
        pltpu.VMEM((CH,), jnp.int32),
        pltpu.VMEM((CH,), jnp.int32),
        pltpu.VMEM((CH, D), jnp.float32),
        pltpu.VMEM((RPT, D), jnp.float32),
        pltpu.VMEM_SHARED((ROWS_PAD, D), jnp.float32),
        pltpu.SemaphoreType.DMA,
    ],
)
def _spmm_sc(xs_hbm, rows_hbm, cols_hbm, zeros_hbm, out_hbm,
             cidx, ridx, gbuf, zb, acc, sem):
    c = lax.axis_index("c")
    s = lax.axis_index("s")
    pltpu.sync_copy(zeros_hbm, zb)
    pltpu.sync_copy(zb, acc.at[pl.ds(s * RPT, RPT)])
    plsc.subcore_barrier()

    def body(j, _):
        off = s * EPT + j * CH
        pltpu.sync_copy(cols_hbm.at[c, pl.ds(off, CH)], cidx)
        pltpu.async_copy(xs_hbm.at[cidx], gbuf, sem).wait()
        pltpu.sync_copy(rows_hbm.at[c, pl.ds(off, CH)], ridx)
        pltpu.sync_copy(gbuf, acc.at[ridx], add=True)
        return 0

    lax.fori_loop(0, NCHUNK, body, 0)
    plsc.subcore_barrier()
    pltpu.sync_copy(acc.at[pl.ds(s * RPT, RPT)], zb)
    pltpu.sync_copy(zb, out_hbm.at[c, pl.ds(s * RPT, RPT)])


def _init_tc_body(deg_ref, ego_ref, dinv_ref, xs_ref):
    d = deg_ref[...]
    dinv = jnp.where(d > 0.0, lax.rsqrt(jnp.maximum(d, 1e-12)), 0.0)
    dinv_ref[...] = dinv
    xs_ref[...] = dinv * ego_ref[...]


def _layer_tc_body(ego_ref, sraw_ref, dinv_ref, w1_ref, w2_ref, acc_ref,
                   ego_out, xs_out, acc_out):
    dinv = dinv_ref[...]
    s = dinv * sraw_ref[0]
    e = ego_ref[...]
    h = jnp.dot(e + s, w1_ref[...], preferred_element_type=jnp.float32,
                precision=lax.Precision.HIGHEST)
    h += jnp.dot(s * e, w2_ref[...], preferred_element_type=jnp.float32,
                 precision=lax.Precision.HIGHEST)
    en = jnp.where(h >= 0.0, h, 0.01 * h)
    ego_out[...] = en
    xs_out[...] = dinv * en
    acc_out[...] = acc_ref[...] + en


_RB = 1000


def _row_spec():
    return pl.BlockSpec((_RB, D), lambda i: (i, 0))


def _sraw_spec():
    return pl.BlockSpec((1, _RB, D), lambda i: (i // 5, i % 5, 0))


def _w_spec():
    return pl.BlockSpec((D, D), lambda i: (0, 0))


_init_tc = pl.pallas_call(
    _init_tc_body,
    grid=(N // _RB,),
    in_specs=[_row_spec(), _row_spec()],
    out_specs=[_row_spec(), _row_spec()],
    out_shape=[
        jax.ShapeDtypeStruct((N, D), jnp.float32),
        jax.ShapeDtypeStruct((N, D), jnp.float32),
    ],
)

_layer_tc = pl.pallas_call(
    _layer_tc_body,
    grid=(N // _RB,),
    in_specs=[_row_spec(), _sraw_spec(), _row_spec(), _w_spec(), _w_spec(),
              _row_spec()],
    out_specs=[_row_spec(), _row_spec(), _row_spec()],
    out_shape=[
        jax.ShapeDtypeStruct((N, D), jnp.float32),
        jax.ShapeDtypeStruct((N, D), jnp.float32),
        jax.ShapeDtypeStruct((N, D), jnp.float32),
    ],
)


def kernel(user_emb, item_emb, edge_index, W1, W2):
    src = edge_index[0]
    dst = edge_index[1]
    npad = E_PAD - E
    pad_r = jnp.full((npad,), PAD_ROW, dtype=jnp.int32)
    pad_c = jnp.zeros((npad,), dtype=jnp.int32)
    rows_all = jnp.stack([
        jnp.concatenate([src, pad_r]),
        jnp.concatenate([dst, pad_r]),
    ])
    cols_all = jnp.stack([
        jnp.concatenate([dst + N_U, pad_c]),
        jnp.concatenate([src, pad_c]),
    ])

    ones16 = jnp.ones((CH, 16), jnp.float32)
    zeros16 = jnp.zeros((RPT, 16), jnp.float32)
    zerosD = jnp.zeros((RPT, D), jnp.float32)

    deg_sc = _deg_sc(rows_all, ones16, zeros16)
    deg_vec = jnp.concatenate([deg_sc[0, :N_U, 0], deg_sc[1, :N_I, 0]])
    deg_b = jnp.broadcast_to(deg_vec[:, None], (N, D))

    ego = jnp.concatenate([user_emb, item_emb], axis=0)
    dinv, xs = _init_tc(deg_b, ego)

    acc = ego
    for k in range(L):
        s_raw = _spmm_sc(xs, rows_all, cols_all, zerosD)
        ego, xs, acc = _layer_tc(ego, s_raw, dinv, W1[k], W2[k], acc)

    mean = acc * 0.25
    return (mean[:N_U], mean[N_U:])


# speedup vs baseline: 20.2970x; 20.2970x over previous
import functools

import jax
import jax.numpy as jnp
from jax import lax
from jax.experimental import pallas as pl
from jax.experimental.pallas import tpu as pltpu
from jax.experimental.pallas import tpu_sc as plsc

N_U = 5000
N_I = 5000
N = 10000
D = 128
L = 3
E = 320000

NC = 2
NS = 16
CH = 128
EPT = 20096
NCHUNK = EPT // CH
E_PAD = EPT * NS
ROWS_PAD = 5120
RPT = ROWS_PAD // NS
PAD_ROW = ROWS_PAD - 1

def _deg_sc_body(rows_hbm, ones_hbm, zeros_hbm, out_hbm, ridx, ones_v, zb, acc):
    c = lax.axis_index("c")
    s = lax.axis_index("s")
    pltpu.sync_copy(zeros_hbm, zb)
    pltpu.sync_copy(zb, acc.at[pl.ds(s * RPT, RPT)])
    pltpu.sync_copy(ones_hbm, ones_v)
    plsc.subcore_barrier()

    def body(j, _):
        off = s * EPT + j * CH
        pltpu.sync_copy(rows_hbm.at[c, pl.ds(off, CH)], ridx)
        pltpu.sync_copy(ones_v, acc.at[ridx], add=True)
        return 0

    lax.fori_loop(0, NCHUNK, body, 0)
    plsc.subcore_barrier()
    pltpu.sync_copy(acc.at[pl.ds(s * RPT, RPT)], zb)
    pltpu.sync_copy(zb, out_hbm.at[c, pl.ds(s * RPT, RPT)])


def _spmm_sc_body(xs_hbm, rows_hbm, cols_hbm, zeros_hbm, out_hbm,
                  cidx, ridx, gbuf, zb, acc, sem):
    c = lax.axis_index("c")
    s = lax.axis_index("s")
    pltpu.sync_copy(zeros_hbm, zb)
    pltpu.sync_copy(zb, acc.at[pl.ds(s * RPT, RPT)])
    plsc.subcore_barrier()

    def body(j, _):
        off = s * EPT + j * CH
        pltpu.sync_copy(cols_hbm.at[c, pl.ds(off, CH)], cidx)
        pltpu.async_copy(xs_hbm.at[cidx], gbuf, sem).wait()
        pltpu.sync_copy(rows_hbm.at[c, pl.ds(off, CH)], ridx)
        pltpu.sync_copy(gbuf, acc.at[ridx], add=True)
        return 0

    lax.fori_loop(0, NCHUNK, body, 0)
    plsc.subcore_barrier()
    pltpu.sync_copy(acc.at[pl.ds(s * RPT, RPT)], zb)
    pltpu.sync_copy(zb, out_hbm.at[c, pl.ds(s * RPT, RPT)])


@functools.lru_cache(maxsize=None)
def _build_sc_kernels():
    mesh = plsc.VectorSubcoreMesh(core_axis_name="c", subcore_axis_name="s")
    deg_sc = pl.kernel(
        _deg_sc_body,
        out_type=jax.ShapeDtypeStruct((NC, ROWS_PAD, D), jnp.float32),
        mesh=mesh,
        scratch_types=[
            pltpu.VMEM((CH,), jnp.int32),
            pltpu.VMEM((CH, D), jnp.float32),
            pltpu.VMEM((RPT, D), jnp.float32),
            pltpu.VMEM_SHARED((ROWS_PAD, D), jnp.float32),
        ],
    )
    spmm_sc = pl.kernel(
        _spmm_sc_body,
        out_type=jax.ShapeDtypeStruct((NC, ROWS_PAD, D), jnp.float32),
        mesh=mesh,
        scratch_types=[
            pltpu.VMEM((CH,), jnp.int32),
            pltpu.VMEM((CH,), jnp.int32),
            pltpu.VMEM((CH, D), jnp.float32),
            pltpu.VMEM((RPT, D), jnp.float32),
            pltpu.VMEM_SHARED((ROWS_PAD, D), jnp.float32),
            pltpu.SemaphoreType.DMA,
        ],
    )
    return deg_sc, spmm_sc


def _init_tc_body(deg_ref, ego_ref, dinv_ref, xs_ref):
    d = deg_ref[...]
    dinv = jnp.where(d > 0.0, lax.rsqrt(jnp.maximum(d, 1e-12)), 0.0)
    dinv_ref[...] = dinv
    xs_ref[...] = dinv * ego_ref[...]


def _layer_tc_body(ego_ref, sraw_ref, dinv_ref, w1_ref, w2_ref, acc_ref,
                   ego_out, xs_out, acc_out):
    dinv = dinv_ref[...]
    s = dinv * sraw_ref[0]
    e = ego_ref[...]
    h = jnp.dot(e + s, w1_ref[...], preferred_element_type=jnp.float32,
                precision=lax.Precision.HIGHEST)
    h += jnp.dot(s * e, w2_ref[...], preferred_element_type=jnp.float32,
                 precision=lax.Precision.HIGHEST)
    en = jnp.where(h >= 0.0, h, 0.01 * h)
    ego_out[...] = en
    xs_out[...] = dinv * en
    acc_out[...] = acc_ref[...] + en


_RB = 1000


def _row_spec():
    return pl.BlockSpec((_RB, D), lambda i: (i, 0))


def _sraw_spec():
    return pl.BlockSpec((1, _RB, D), lambda i: (i // 5, i % 5, 0))


def _w_spec():
    return pl.BlockSpec((D, D), lambda i: (0, 0))


_init_tc = pl.pallas_call(
    _init_tc_body,
    grid=(N // _RB,),
    in_specs=[_row_spec(), _row_spec()],
    out_specs=[_row_spec(), _row_spec()],
    out_shape=[
        jax.ShapeDtypeStruct((N, D), jnp.float32),
        jax.ShapeDtypeStruct((N, D), jnp.float32),
    ],
)

_layer_tc = pl.pallas_call(
    _layer_tc_body,
    grid=(N // _RB,),
    in_specs=[_row_spec(), _sraw_spec(), _row_spec(), _w_spec(), _w_spec(),
              _row_spec()],
    out_specs=[_row_spec(), _row_spec(), _row_spec()],
    out_shape=[
        jax.ShapeDtypeStruct((N, D), jnp.float32),
        jax.ShapeDtypeStruct((N, D), jnp.float32),
        jax.ShapeDtypeStruct((N, D), jnp.float32),
    ],
)


def kernel(user_emb, item_emb, edge_index, W1, W2):
    src = edge_index[0]
    dst = edge_index[1]
    npad = E_PAD - E
    pad_r = jnp.full((npad,), PAD_ROW, dtype=jnp.int32)
    pad_c = jnp.zeros((npad,), dtype=jnp.int32)
    rows_all = jnp.stack([
        jnp.concatenate([src, pad_r]),
        jnp.concatenate([dst, pad_r]),
    ])
    cols_all = jnp.stack([
        jnp.concatenate([dst + N_U, pad_c]),
        jnp.concatenate([src, pad_c]),
    ])

    onesD = jnp.ones((CH, D), jnp.float32)
    zerosD = jnp.zeros((RPT, D), jnp.float32)

    _deg_sc, _spmm_sc = _build_sc_kernels()
    deg_sc = _deg_sc(rows_all, onesD, zerosD)
    deg_b = jnp.concatenate([deg_sc[0, :N_U], deg_sc[1, :N_I]], axis=0)

    ego = jnp.concatenate([user_emb, item_emb], axis=0)
    dinv, xs = _init_tc(deg_b, ego)

    acc = ego
    for k in range(L):
        s_raw = _spmm_sc(xs, rows_all, cols_all, zerosD)
        ego, xs, acc = _layer_tc(ego, s_raw, dinv, W1[k], W2[k], acc)

    mean = acc * 0.25
    return (mean[:N_U], mean[N_U:])


# speedup vs baseline: 34.9481x; 1.7218x over previous
import functools

import jax
import jax.numpy as jnp
from jax import lax
from jax.experimental import pallas as pl
from jax.experimental.pallas import tpu as pltpu
from jax.experimental.pallas import tpu_sc as plsc

N_U = 5000
N_I = 5000
N = 10000
D = 128
L = 3
E = 320000

NC = 2
NS = 16
CH = 128
EPT = 20096
NCHUNK = EPT // CH
E_PAD = EPT * NS
ROWS_PAD = 5120
RPT = ROWS_PAD // NS
PAD_ROW = ROWS_PAD - 1
ZB = 80

def _deg_sc_body(edges_hbm, ones_hbm, zeros_hbm, out_hbm, idxall, ones_v, zb, acc):
    c = lax.axis_index("c")
    s = lax.axis_index("s")
    pltpu.sync_copy(edges_hbm.at[c, s, :, pl.ds(1, 1), :], idxall)
    pltpu.sync_copy(ones_hbm, ones_v)
    pltpu.sync_copy(zeros_hbm, zb)
    for p in range(RPT // ZB):
        pltpu.sync_copy(zb, acc.at[pl.ds(s * RPT + p * ZB, ZB)])
    plsc.subcore_barrier()

    def body(j, _):
        pltpu.sync_copy(ones_v, acc.at[idxall.at[j, 0]], add=True)
        return 0

    lax.fori_loop(0, NCHUNK, body, 0)
    plsc.subcore_barrier()
    for p in range(RPT // ZB):
        pltpu.sync_copy(acc.at[pl.ds(s * RPT + p * ZB, ZB)], zb)
        pltpu.sync_copy(zb, out_hbm.at[c, pl.ds(s * RPT + p * ZB, ZB)])


def _spmm_sc_body(xs_hbm, edges_hbm, zeros_hbm, out_hbm,
                  idxall, gbuf, zb, acc, gsem):
    c = lax.axis_index("c")
    s = lax.axis_index("s")
    pltpu.sync_copy(edges_hbm.at[c, s], idxall)
    pltpu.sync_copy(zeros_hbm, zb)
    for p in range(RPT // ZB):
        pltpu.sync_copy(zb, acc.at[pl.ds(s * RPT + p * ZB, ZB)])
    plsc.subcore_barrier()

    pltpu.async_copy(xs_hbm.at[idxall.at[0, 0]], gbuf.at[0], gsem)

    def body(j, _):
        b = lax.rem(j, 2)
        nb = lax.rem(j + 1, 2)

        @pl.when(j + 1 < NCHUNK)
        def _prefetch():
            pltpu.async_copy(xs_hbm.at[idxall.at[j + 1, 0]], gbuf.at[nb], gsem)

        pltpu.make_async_copy(xs_hbm.at[idxall.at[j, 0]], gbuf.at[b], gsem).wait()
        pltpu.sync_copy(gbuf.at[b], acc.at[idxall.at[j, 1]], add=True)
        return 0

    lax.fori_loop(0, NCHUNK, body, 0)
    plsc.subcore_barrier()
    for p in range(RPT // ZB):
        pltpu.sync_copy(acc.at[pl.ds(s * RPT + p * ZB, ZB)], zb)
        pltpu.sync_copy(zb, out_hbm.at[c, pl.ds(s * RPT + p * ZB, ZB)])


@functools.lru_cache(maxsize=None)
def _build_sc_kernels():
    mesh = plsc.VectorSubcoreMesh(core_axis_name="c", subcore_axis_name="s")
    deg_sc = pl.kernel(
        _deg_sc_body,
        out_type=jax.ShapeDtypeStruct((NC, ROWS_PAD, D), jnp.float32),
        mesh=mesh,
        scratch_types=[
            pltpu.VMEM((NCHUNK, 1, CH), jnp.int32),
            pltpu.VMEM((CH, D), jnp.float32),
            pltpu.VMEM((ZB, D), jnp.float32),
            pltpu.VMEM_SHARED((ROWS_PAD, D), jnp.float32),
        ],
    )
    spmm_sc = pl.kernel(
        _spmm_sc_body,
        out_type=jax.ShapeDtypeStruct((NC, ROWS_PAD, D), jnp.float32),
        mesh=mesh,
        scratch_types=[
            pltpu.VMEM((NCHUNK, 2, CH), jnp.int32),
            pltpu.VMEM((2, CH, D), jnp.float32),
            pltpu.VMEM((ZB, D), jnp.float32),
            pltpu.VMEM_SHARED((ROWS_PAD, D), jnp.float32),
            pltpu.SemaphoreType.DMA,
        ],
    )
    return deg_sc, spmm_sc


def _init_tc_body(deg_ref, ego_ref, dinv_ref, xs_ref):
    d = deg_ref[...]
    dinv = jnp.where(d > 0.0, lax.rsqrt(jnp.maximum(d, 1e-12)), 0.0)
    dinv_ref[...] = dinv
    xs_ref[...] = dinv * ego_ref[...]


def _layer_tc_body(ego_ref, sraw_ref, dinv_ref, w1_ref, w2_ref, acc_ref,
                   ego_out, xs_out, acc_out):
    dinv = dinv_ref[...]
    s = dinv * sraw_ref[0]
    e = ego_ref[...]
    h = jnp.dot(e + s, w1_ref[...], preferred_element_type=jnp.float32,
                precision=lax.Precision.HIGHEST)
    h += jnp.dot(s * e, w2_ref[...], preferred_element_type=jnp.float32,
                 precision=lax.Precision.HIGHEST)
    en = jnp.where(h >= 0.0, h, 0.01 * h)
    ego_out[...] = en
    xs_out[...] = dinv * en
    acc_out[...] = acc_ref[...] + en


_RB = 1000


def _row_spec():
    return pl.BlockSpec((_RB, D), lambda i: (i, 0))


def _sraw_spec():
    return pl.BlockSpec((1, _RB, D), lambda i: (i // 5, i % 5, 0))


def _w_spec():
    return pl.BlockSpec((D, D), lambda i: (0, 0))


_init_tc = pl.pallas_call(
    _init_tc_body,
    grid=(N // _RB,),
    in_specs=[_row_spec(), _row_spec()],
    out_specs=[_row_spec(), _row_spec()],
    out_shape=[
        jax.ShapeDtypeStruct((N, D), jnp.float32),
        jax.ShapeDtypeStruct((N, D), jnp.float32),
    ],
)

_layer_tc = pl.pallas_call(
    _layer_tc_body,
    grid=(N // _RB,),
    in_specs=[_row_spec(), _sraw_spec(), _row_spec(), _w_spec(), _w_spec(),
              _row_spec()],
    out_specs=[_row_spec(), _row_spec(), _row_spec()],
    out_shape=[
        jax.ShapeDtypeStruct((N, D), jnp.float32),
        jax.ShapeDtypeStruct((N, D), jnp.float32),
        jax.ShapeDtypeStruct((N, D), jnp.float32),
    ],
)


def kernel(user_emb, item_emb, edge_index, W1, W2):
    src = edge_index[0]
    dst = edge_index[1]
    npad = E_PAD - E
    pad_r = jnp.full((npad,), PAD_ROW, dtype=jnp.int32)
    pad_c = jnp.zeros((npad,), dtype=jnp.int32)
    rows_all = jnp.stack([
        jnp.concatenate([src, pad_r]),
        jnp.concatenate([dst, pad_r]),
    ]).reshape(NC, NS, NCHUNK, 1, CH)
    cols_all = jnp.stack([
        jnp.concatenate([dst + N_U, pad_c]),
        jnp.concatenate([src, pad_c]),
    ]).reshape(NC, NS, NCHUNK, 1, CH)
    edges = jnp.concatenate([cols_all, rows_all], axis=3)

    onesD = jnp.ones((CH, D), jnp.float32)
    zerosD = jnp.zeros((ZB, D), jnp.float32)

    _deg_sc, _spmm_sc = _build_sc_kernels()
    deg_sc = _deg_sc(edges, onesD, zerosD)
    deg_b = jnp.concatenate([deg_sc[0, :N_U], deg_sc[1, :N_I]], axis=0)

    ego = jnp.concatenate([user_emb, item_emb], axis=0)
    dinv, xs = _init_tc(deg_b, ego)

    acc = ego
    for k in range(L):
        s_raw = _spmm_sc(xs, edges, zerosD)
        ego, xs, acc = _layer_tc(ego, s_raw, dinv, W1[k], W2[k], acc)

    mean = acc * 0.25
    return (mean[:N_U], mean[N_U:])


# speedup vs baseline: 34.9999x; 1.0015x over previous
import functools

import jax
import jax.numpy as jnp
from jax import lax
from jax.experimental import pallas as pl
from jax.experimental.pallas import tpu as pltpu
from jax.experimental.pallas import tpu_sc as plsc

N_U = 5000
N_I = 5000
N = 10000
D = 128
L = 3
E = 320000

NC = 2
NS = 16
CH = 128
EPT = 20096
NCHUNK = EPT // CH
E_PAD = EPT * NS
ROWS_PAD = 5120
RPT = ROWS_PAD // NS
PAD_ROW = ROWS_PAD - 1
ZB = 80

K_DEG = 4


def _deg_sc_body(edges_hbm, ones_hbm, zeros_hbm, out_hbm, idxall, ones_v, zb,
                 acc, ssem):
    c = lax.axis_index("c")
    s = lax.axis_index("s")
    pltpu.sync_copy(edges_hbm.at[c, s, :, pl.ds(1, 1), :], idxall)
    pltpu.sync_copy(ones_hbm, ones_v)
    pltpu.sync_copy(zeros_hbm, zb)
    for p in range(RPT // ZB):
        pltpu.sync_copy(zb, acc.at[pl.ds(s * RPT + p * ZB, ZB)])
    plsc.subcore_barrier()

    def body(j, _):
        @pl.when(j >= K_DEG)
        def _drain():
            pltpu.make_async_copy(ones_v, acc.at[idxall.at[j - K_DEG, 0]],
                                  ssem).wait()

        pltpu.async_copy(ones_v, acc.at[idxall.at[j, 0]], ssem, add=True)
        return 0

    lax.fori_loop(0, NCHUNK, body, 0)
    for t in range(K_DEG):
        pltpu.make_async_copy(ones_v, acc.at[idxall.at[NCHUNK - K_DEG + t, 0]],
                              ssem).wait()
    plsc.subcore_barrier()
    for p in range(RPT // ZB):
        pltpu.sync_copy(acc.at[pl.ds(s * RPT + p * ZB, ZB)], zb)
        pltpu.sync_copy(zb, out_hbm.at[c, pl.ds(s * RPT + p * ZB, ZB)])


def _spmm_sc_body(xs_hbm, edges_hbm, zeros_hbm, out_hbm,
                  idxall, gbuf, zb, acc, gsem, ssem):
    c = lax.axis_index("c")
    s = lax.axis_index("s")
    pltpu.sync_copy(edges_hbm.at[c, s], idxall)
    pltpu.sync_copy(zeros_hbm, zb)
    for p in range(RPT // ZB):
        pltpu.sync_copy(zb, acc.at[pl.ds(s * RPT + p * ZB, ZB)])
    plsc.subcore_barrier()

    pltpu.async_copy(xs_hbm.at[idxall.at[0, 0]], gbuf.at[0], gsem)

    def body(j, _):
        b = lax.rem(j, 2)
        nb = lax.rem(j + 1, 2)

        @pl.when(j >= 1)
        def _drain():
            pltpu.make_async_copy(gbuf.at[nb], acc.at[idxall.at[j - 1, 1]],
                                  ssem).wait()

        @pl.when(j + 1 < NCHUNK)
        def _prefetch():
            pltpu.async_copy(xs_hbm.at[idxall.at[j + 1, 0]], gbuf.at[nb], gsem)

        pltpu.make_async_copy(xs_hbm.at[idxall.at[j, 0]], gbuf.at[b], gsem).wait()
        pltpu.async_copy(gbuf.at[b], acc.at[idxall.at[j, 1]], ssem, add=True)
        return 0

    lax.fori_loop(0, NCHUNK, body, 0)
    pltpu.make_async_copy(gbuf.at[0], acc.at[idxall.at[NCHUNK - 1, 1]],
                          ssem).wait()
    plsc.subcore_barrier()
    for p in range(RPT // ZB):
        pltpu.sync_copy(acc.at[pl.ds(s * RPT + p * ZB, ZB)], zb)
        pltpu.sync_copy(zb, out_hbm.at[c, pl.ds(s * RPT + p * ZB, ZB)])


@functools.lru_cache(maxsize=None)
def _build_sc_kernels():
    mesh = plsc.VectorSubcoreMesh(core_axis_name="c", subcore_axis_name="s")
    deg_sc = pl.kernel(
        _deg_sc_body,
        out_type=jax.ShapeDtypeStruct((NC, ROWS_PAD, D), jnp.float32),
        mesh=mesh,
        scratch_types=[
            pltpu.VMEM((NCHUNK, 1, CH), jnp.int32),
            pltpu.VMEM((CH, D), jnp.float32),
            pltpu.VMEM((ZB, D), jnp.float32),
            pltpu.VMEM_SHARED((ROWS_PAD, D), jnp.float32),
            pltpu.SemaphoreType.DMA,
        ],
    )
    spmm_sc = pl.kernel(
        _spmm_sc_body,
        out_type=jax.ShapeDtypeStruct((NC, ROWS_PAD, D), jnp.float32),
        mesh=mesh,
        scratch_types=[
            pltpu.VMEM((NCHUNK, 2, CH), jnp.int32),
            pltpu.VMEM((2, CH, D), jnp.float32),
            pltpu.VMEM((ZB, D), jnp.float32),
            pltpu.VMEM_SHARED((ROWS_PAD, D), jnp.float32),
            pltpu.SemaphoreType.DMA,
            pltpu.SemaphoreType.DMA,
        ],
    )
    return deg_sc, spmm_sc


def _init_tc_body(deg_ref, ego_ref, dinv_ref, xs_ref):
    d = deg_ref[...]
    dinv = jnp.where(d > 0.0, lax.rsqrt(jnp.maximum(d, 1e-12)), 0.0)
    dinv_ref[...] = dinv
    xs_ref[...] = dinv * ego_ref[...]


def _layer_tc_body(ego_ref, sraw_ref, dinv_ref, w1_ref, w2_ref, acc_ref,
                   ego_out, xs_out, acc_out):
    dinv = dinv_ref[...]
    s = dinv * sraw_ref[0]
    e = ego_ref[...]
    h = jnp.dot(e + s, w1_ref[...], preferred_element_type=jnp.float32,
                precision=lax.Precision.HIGHEST)
    h += jnp.dot(s * e, w2_ref[...], preferred_element_type=jnp.float32,
                 precision=lax.Precision.HIGHEST)
    en = jnp.where(h >= 0.0, h, 0.01 * h)
    ego_out[...] = en
    xs_out[...] = dinv * en
    acc_out[...] = acc_ref[...] + en


_RB = 1000


def _row_spec():
    return pl.BlockSpec((_RB, D), lambda i: (i, 0))


def _sraw_spec():
    return pl.BlockSpec((1, _RB, D), lambda i: (i // 5, i % 5, 0))


def _w_spec():
    return pl.BlockSpec((D, D), lambda i: (0, 0))


_init_tc = pl.pallas_call(
    _init_tc_body,
    grid=(N // _RB,),
    in_specs=[_row_spec(), _row_spec()],
    out_specs=[_row_spec(), _row_spec()],
    out_shape=[
        jax.ShapeDtypeStruct((N, D), jnp.float32),
        jax.ShapeDtypeStruct((N, D), jnp.float32),
    ],
)

_layer_tc = pl.pallas_call(
    _layer_tc_body,
    grid=(N // _RB,),
    in_specs=[_row_spec(), _sraw_spec(), _row_spec(), _w_spec(), _w_spec(),
              _row_spec()],
    out_specs=[_row_spec(), _row_spec(), _row_spec()],
    out_shape=[
        jax.ShapeDtypeStruct((N, D), jnp.float32),
        jax.ShapeDtypeStruct((N, D), jnp.float32),
        jax.ShapeDtypeStruct((N, D), jnp.float32),
    ],
)


def kernel(user_emb, item_emb, edge_index, W1, W2):
    src = edge_index[0]
    dst = edge_index[1]
    npad = E_PAD - E
    pad_r = jnp.full((npad,), PAD_ROW, dtype=jnp.int32)
    pad_c = jnp.zeros((npad,), dtype=jnp.int32)
    rows_all = jnp.stack([
        jnp.concatenate([src, pad_r]),
        jnp.concatenate([dst, pad_r]),
    ]).reshape(NC, NS, NCHUNK, 1, CH)
    cols_all = jnp.stack([
        jnp.concatenate([dst + N_U, pad_c]),
        jnp.concatenate([src, pad_c]),
    ]).reshape(NC, NS, NCHUNK, 1, CH)
    edges = jnp.concatenate([cols_all, rows_all], axis=3)

    onesD = jnp.ones((CH, D), jnp.float32)
    zerosD = jnp.zeros((ZB, D), jnp.float32)

    _deg_sc, _spmm_sc = _build_sc_kernels()
    deg_sc = _deg_sc(edges, onesD, zerosD)
    deg_b = jnp.concatenate([deg_sc[0, :N_U], deg_sc[1, :N_I]], axis=0)

    ego = jnp.concatenate([user_emb, item_emb], axis=0)
    dinv, xs = _init_tc(deg_b, ego)

    acc = ego
    for k in range(L):
        s_raw = _spmm_sc(xs, edges, zerosD)
        ego, xs, acc = _layer_tc(ego, s_raw, dinv, W1[k], W2[k], acc)

    mean = acc * 0.25
    return (mean[:N_U], mean[N_U:])


# speedup vs baseline: 36.8073x; 1.0516x over previous
import functools

import jax
import jax.numpy as jnp
from jax import lax
from jax.experimental import pallas as pl
from jax.experimental.pallas import tpu as pltpu
from jax.experimental.pallas import tpu_sc as plsc

N_U = 5000
N_I = 5000
N = 10000
D = 128
L = 3
E = 320000

NC = 2
NS = 16
CH = 128
KG = 16
RING = 3
NCHUNK = 160
NGRP = NCHUNK // KG
EPT = NCHUNK * CH
E_PAD = EPT * NS
ROWS_PAD = 5120
RPT = ROWS_PAD // NS
PAD_ROW = ROWS_PAD - 1

K_DEG = 4


def _deg_sc_body(edges_hbm, ones_hbm, zeros_hbm, out_hbm, idxall, ones_v,
                 acc, ssem):
    c = lax.axis_index("c")
    s = lax.axis_index("s")
    pltpu.sync_copy(edges_hbm.at[c, s, :, pl.ds(1, 1), :], idxall)
    pltpu.sync_copy(ones_hbm, ones_v)
    pltpu.sync_copy(zeros_hbm, acc.at[pl.ds(s * RPT, RPT)])
    plsc.subcore_barrier()

    def body(j, _):
        @pl.when(j >= K_DEG)
        def _drain():
            pltpu.make_async_copy(ones_v, acc.at[idxall.at[j - K_DEG, 0]],
                                  ssem).wait()

        pltpu.async_copy(ones_v, acc.at[idxall.at[j, 0]], ssem, add=True)
        return 0

    lax.fori_loop(0, NCHUNK, body, 0)
    for t in range(K_DEG):
        pltpu.make_async_copy(ones_v, acc.at[idxall.at[NCHUNK - K_DEG + t, 0]],
                              ssem).wait()
    plsc.subcore_barrier()
    pltpu.sync_copy(acc.at[pl.ds(s * RPT, RPT)],
                    out_hbm.at[c, pl.ds(s * RPT, RPT)])


def _spmm_sc_body(xs_hbm, edges_hbm, zeros_hbm, out_hbm,
                  ring, gbuf, stage, acc, gsem, ssem, isem):
    c = lax.axis_index("c")
    s = lax.axis_index("s")
    pltpu.sync_copy(xs_hbm.at[1 - c, pl.ds(s * RPT, RPT)],
                    stage.at[pl.ds(s * RPT, RPT)])
    pltpu.sync_copy(zeros_hbm, acc.at[pl.ds(s * RPT, RPT)])
    pltpu.sync_copy(edges_hbm.at[c, s, pl.ds(0, KG)], ring.at[0])
    pltpu.async_copy(edges_hbm.at[c, s, pl.ds(KG, KG)], ring.at[1], isem)
    plsc.subcore_barrier()

    pltpu.async_copy(stage.at[ring.at[0, 0, 0]], gbuf.at[0], gsem)

    def body(j, _):
        b = lax.rem(j, 2)
        nb = lax.rem(j + 1, 2)
        g = lax.div(j, KG)
        k = lax.rem(j, KG)

        @pl.when(j >= 1)
        def _drain():
            pltpu.make_async_copy(
                gbuf.at[nb],
                acc.at[ring.at[lax.rem(lax.div(j - 1, KG), RING),
                               lax.rem(j - 1, KG), 1]],
                ssem).wait()

        @pl.when(jnp.logical_and(k == 0, g + 1 < NGRP))
        def _ringwait():
            pltpu.make_async_copy(edges_hbm.at[c, s, pl.ds(0, KG)],
                                  ring.at[0], isem).wait()

        @pl.when(jnp.logical_and(k == 2, g + 2 < NGRP))
        def _ringfill():
            pltpu.async_copy(edges_hbm.at[c, s, pl.ds((g + 2) * KG, KG)],
                             ring.at[lax.rem(g + 2, RING)], isem)

        @pl.when(j + 1 < NCHUNK)
        def _prefetch():
            pltpu.async_copy(
                stage.at[ring.at[lax.rem(lax.div(j + 1, KG), RING),
                                 lax.rem(j + 1, KG), 0]],
                gbuf.at[nb], gsem)

        pltpu.make_async_copy(stage.at[ring.at[lax.rem(g, RING), k, 0]],
                              gbuf.at[b], gsem).wait()
        pltpu.async_copy(gbuf.at[b],
                         acc.at[ring.at[lax.rem(g, RING), k, 1]],
                         ssem, add=True)
        return 0

    lax.fori_loop(0, NCHUNK, body, 0)
    pltpu.make_async_copy(gbuf.at[0], acc.at[ring.at[0, 0, 1]], ssem).wait()
    plsc.subcore_barrier()
    pltpu.sync_copy(acc.at[pl.ds(s * RPT, RPT)],
                    out_hbm.at[c, pl.ds(s * RPT, RPT)])


@functools.lru_cache(maxsize=None)
def _build_sc_kernels():
    mesh = plsc.VectorSubcoreMesh(core_axis_name="c", subcore_axis_name="s")
    deg_sc = pl.kernel(
        _deg_sc_body,
        out_type=jax.ShapeDtypeStruct((NC, ROWS_PAD, D), jnp.float32),
        mesh=mesh,
        scratch_types=[
            pltpu.VMEM((NCHUNK, 1, CH), jnp.int32),
            pltpu.VMEM((CH, D), jnp.float32),
            pltpu.VMEM_SHARED((ROWS_PAD, D), jnp.float32),
            pltpu.SemaphoreType.DMA,
        ],
    )
    spmm_sc = pl.kernel(
        _spmm_sc_body,
        out_type=jax.ShapeDtypeStruct((NC, ROWS_PAD, D), jnp.float32),
        mesh=mesh,
        scratch_types=[
            pltpu.VMEM((RING, KG, 2, CH), jnp.int32),
            pltpu.VMEM((2, CH, D), jnp.float32),
            pltpu.VMEM_SHARED((ROWS_PAD, D), jnp.float32),
            pltpu.VMEM_SHARED((ROWS_PAD, D), jnp.float32),
            pltpu.SemaphoreType.DMA,
            pltpu.SemaphoreType.DMA,
            pltpu.SemaphoreType.DMA,
        ],
    )
    return deg_sc, spmm_sc


def _init_tc_body(deg_ref, ego_ref, dinv_ref, xs_ref):
    d = deg_ref[...]
    dinv = jnp.where(d > 0.0, lax.rsqrt(jnp.maximum(d, 1e-12)), 0.0)
    dinv_ref[...] = dinv
    xs_ref[0] = dinv * ego_ref[...]


def _layer_tc_body(ego_ref, sraw_ref, dinv_ref, w1_ref, w2_ref, acc_ref,
                   ego_out, xs_out, acc_out):
    dinv = dinv_ref[...]
    s = dinv * sraw_ref[0]
    e = ego_ref[...]
    h = jnp.dot(e + s, w1_ref[...], preferred_element_type=jnp.float32,
                precision=lax.Precision.HIGHEST)
    h += jnp.dot(s * e, w2_ref[...], preferred_element_type=jnp.float32,
                 precision=lax.Precision.HIGHEST)
    en = jnp.where(h >= 0.0, h, 0.01 * h)
    ego_out[...] = en
    xs_out[0] = dinv * en
    acc_out[...] = acc_ref[...] + en


_RB = 1000


def _row_spec():
    return pl.BlockSpec((_RB, D), lambda i: (i, 0))


def _half_spec():
    return pl.BlockSpec((1, _RB, D), lambda i: (i // 5, i % 5, 0))


def _w_spec():
    return pl.BlockSpec((D, D), lambda i: (0, 0))


_init_tc = pl.pallas_call(
    _init_tc_body,
    grid=(N // _RB,),
    in_specs=[_row_spec(), _row_spec()],
    out_specs=[_row_spec(), _half_spec()],
    out_shape=[
        jax.ShapeDtypeStruct((N, D), jnp.float32),
        jax.ShapeDtypeStruct((NC, ROWS_PAD, D), jnp.float32),
    ],
)

_layer_tc = pl.pallas_call(
    _layer_tc_body,
    grid=(N // _RB,),
    in_specs=[_row_spec(), _half_spec(), _row_spec(), _w_spec(), _w_spec(),
              _row_spec()],
    out_specs=[_row_spec(), _half_spec(), _row_spec()],
    out_shape=[
        jax.ShapeDtypeStruct((N, D), jnp.float32),
        jax.ShapeDtypeStruct((NC, ROWS_PAD, D), jnp.float32),
        jax.ShapeDtypeStruct((N, D), jnp.float32),
    ],
)


def kernel(user_emb, item_emb, edge_index, W1, W2):
    src = edge_index[0]
    dst = edge_index[1]
    npad = E_PAD - E
    pad_r = jnp.full((npad,), PAD_ROW, dtype=jnp.int32)
    pad_c = jnp.zeros((npad,), dtype=jnp.int32)
    rows_all = jnp.stack([
        jnp.concatenate([src, pad_r]),
        jnp.concatenate([dst, pad_r]),
    ]).reshape(NC, NS, NCHUNK, 1, CH)
    cols_all = jnp.stack([
        jnp.concatenate([dst, pad_c]),
        jnp.concatenate([src, pad_c]),
    ]).reshape(NC, NS, NCHUNK, 1, CH)
    edges = jnp.concatenate([cols_all, rows_all], axis=3)

    onesD = jnp.ones((CH, D), jnp.float32)
    zerosD = jnp.zeros((RPT, D), jnp.float32)

    _deg_sc, _spmm_sc = _build_sc_kernels()
    deg_sc = _deg_sc(edges, onesD, zerosD)
    deg_b = jnp.concatenate([deg_sc[0, :N_U], deg_sc[1, :N_I]], axis=0)

    ego = jnp.concatenate([user_emb, item_emb], axis=0)
    dinv, xs = _init_tc(deg_b, ego)

    acc = ego
    for k in range(L):
        s_raw = _spmm_sc(xs, edges, zerosD)
        ego, xs, acc = _layer_tc(ego, s_raw, dinv, W1[k], W2[k], acc)

    mean = acc * 0.25
    return (mean[:N_U], mean[N_U:])


# speedup vs baseline: 37.1316x; 1.0088x over previous
import functools

import jax
import jax.numpy as jnp
from jax import lax
from jax.experimental import pallas as pl
from jax.experimental.pallas import tpu as pltpu
from jax.experimental.pallas import tpu_sc as plsc

N_U = 5000
N_I = 5000
N = 10000
D = 128
L = 3
E = 320000

NC = 2
NS = 16
CH = 128
KG = 16
RING = 3
NCHUNK = 160
NGRP = NCHUNK // KG
EPT = NCHUNK * CH
E_PAD = EPT * NS
ROWS_PAD = 5120
RPT = ROWS_PAD // NS
PAD_ROW = ROWS_PAD - 1

K_DEG = 4


def _deg_sc_body(edges_hbm, ones_hbm, zeros_hbm, out_hbm, idxall, ones_v,
                 acc, ssem):
    c = lax.axis_index("c")
    s = lax.axis_index("s")
    pltpu.sync_copy(edges_hbm.at[c, s, :, pl.ds(1, 1), :], idxall)
    pltpu.sync_copy(ones_hbm, ones_v)
    pltpu.sync_copy(zeros_hbm, acc.at[pl.ds(s * RPT, RPT)])
    plsc.subcore_barrier()

    def body(j, _):
        @pl.when(j >= K_DEG)
        def _drain():
            pltpu.make_async_copy(ones_v, acc.at[idxall.at[j - K_DEG, 0]],
                                  ssem).wait()

        pltpu.async_copy(ones_v, acc.at[idxall.at[j, 0]], ssem, add=True)
        return 0

    lax.fori_loop(0, NCHUNK, body, 0)
    for t in range(K_DEG):
        pltpu.make_async_copy(ones_v, acc.at[idxall.at[NCHUNK - K_DEG + t, 0]],
                              ssem).wait()
    plsc.subcore_barrier()
    pltpu.sync_copy(acc.at[pl.ds(s * RPT, RPT)],
                    out_hbm.at[c, pl.ds(s * RPT, RPT)])


def _spmm_sc_body(xs_hbm, edges_hbm, zeros_hbm, out_hbm,
                  ring, gbuf, stage, acc, gsem, ssem, isem):
    c = lax.axis_index("c")
    s = lax.axis_index("s")
    pltpu.sync_copy(xs_hbm.at[1 - c, pl.ds(s * RPT, RPT)],
                    stage.at[pl.ds(s * RPT, RPT)])
    pltpu.sync_copy(zeros_hbm, acc.at[pl.ds(s * RPT, RPT)])
    pltpu.sync_copy(edges_hbm.at[c, s, pl.ds(0, KG)], ring.at[0])
    pltpu.async_copy(edges_hbm.at[c, s, pl.ds(KG, KG)], ring.at[1], isem)
    plsc.subcore_barrier()

    pltpu.async_copy(stage.at[ring.at[0, 0, 0]], gbuf.at[0], gsem)

    def outer(g, _):
        slot = lax.rem(g, RING)
        slot_next = lax.rem(g + 1, RING)
        slot_prev = lax.rem(g + 2, RING)

        def body(k, _):
            b = lax.rem(k, 2)
            nb = lax.rem(k + 1, 2)

            @pl.when(jnp.logical_and(k == 0, g >= 1))
            def _drain0():
                pltpu.make_async_copy(gbuf.at[nb],
                                      acc.at[ring.at[slot_prev, KG - 1, 1]],
                                      ssem).wait()

            @pl.when(k >= 1)
            def _drain():
                pltpu.make_async_copy(gbuf.at[nb],
                                      acc.at[ring.at[slot, k - 1, 1]],
                                      ssem).wait()

            @pl.when(jnp.logical_and(k == 0, g + 1 < NGRP))
            def _ringwait():
                pltpu.make_async_copy(edges_hbm.at[c, s, pl.ds(0, KG)],
                                      ring.at[0], isem).wait()

            @pl.when(jnp.logical_and(k == 2, g + 2 < NGRP))
            def _ringfill():
                pltpu.async_copy(edges_hbm.at[c, s, pl.ds((g + 2) * KG, KG)],
                                 ring.at[slot_prev], isem)

            @pl.when(k < KG - 1)
            def _prefetch():
                pltpu.async_copy(stage.at[ring.at[slot, k + 1, 0]],
                                 gbuf.at[nb], gsem)

            @pl.when(jnp.logical_and(k == KG - 1, g + 1 < NGRP))
            def _prefetch_x():
                pltpu.async_copy(stage.at[ring.at[slot_next, 0, 0]],
                                 gbuf.at[nb], gsem)

            pltpu.make_async_copy(stage.at[ring.at[slot, k, 0]],
                                  gbuf.at[b], gsem).wait()
            pltpu.async_copy(gbuf.at[b], acc.at[ring.at[slot, k, 1]],
                             ssem, add=True)
            return 0

        lax.fori_loop(0, KG, body, 0)
        return 0

    lax.fori_loop(0, NGRP, outer, 0)
    pltpu.make_async_copy(gbuf.at[0], acc.at[ring.at[0, 0, 1]], ssem).wait()
    plsc.subcore_barrier()
    pltpu.sync_copy(acc.at[pl.ds(s * RPT, RPT)],
                    out_hbm.at[c, pl.ds(s * RPT, RPT)])


@functools.lru_cache(maxsize=None)
def _build_sc_kernels():
    mesh = plsc.VectorSubcoreMesh(core_axis_name="c", subcore_axis_name="s")
    deg_sc = pl.kernel(
        _deg_sc_body,
        out_type=jax.ShapeDtypeStruct((NC, ROWS_PAD, D), jnp.float32),
        mesh=mesh,
        scratch_types=[
            pltpu.VMEM((NCHUNK, 1, CH), jnp.int32),
            pltpu.VMEM((CH, D), jnp.float32),
            pltpu.VMEM_SHARED((ROWS_PAD, D), jnp.float32),
            pltpu.SemaphoreType.DMA,
        ],
    )
    spmm_sc = pl.kernel(
        _spmm_sc_body,
        out_type=jax.ShapeDtypeStruct((NC, ROWS_PAD, D), jnp.float32),
        mesh=mesh,
        scratch_types=[
            pltpu.VMEM((RING, KG, 2, CH), jnp.int32),
            pltpu.VMEM((2, CH, D), jnp.float32),
            pltpu.VMEM_SHARED((ROWS_PAD, D), jnp.float32),
            pltpu.VMEM_SHARED((ROWS_PAD, D), jnp.float32),
            pltpu.SemaphoreType.DMA,
            pltpu.SemaphoreType.DMA,
            pltpu.SemaphoreType.DMA,
        ],
    )
    return deg_sc, spmm_sc


def _init_tc_body(deg_ref, ego_ref, dinv_ref, xs_ref):
    d = deg_ref[0]
    dinv = jnp.where(d > 0.0, lax.rsqrt(jnp.maximum(d, 1e-12)), 0.0)
    dinv_ref[...] = dinv
    xs_ref[0] = dinv * ego_ref[...]


def _layer_tc_body(ego_ref, sraw_ref, dinv_ref, w1_ref, w2_ref, acc_ref,
                   ego_out, xs_out, acc_out):
    dinv = dinv_ref[...]
    s = dinv * sraw_ref[0]
    e = ego_ref[...]
    h = jnp.dot(e + s, w1_ref[...], preferred_element_type=jnp.float32,
                precision=lax.Precision.HIGHEST)
    h += jnp.dot(s * e, w2_ref[...], preferred_element_type=jnp.float32,
                 precision=lax.Precision.HIGHEST)
    en = jnp.where(h >= 0.0, h, 0.01 * h)
    ego_out[...] = en
    xs_out[0] = dinv * en
    acc_out[...] = acc_ref[...] + en


_RB = 1000


def _row_spec():
    return pl.BlockSpec((_RB, D), lambda i: (i, 0))


def _half_spec():
    return pl.BlockSpec((1, _RB, D), lambda i: (i // 5, i % 5, 0))


def _w_spec():
    return pl.BlockSpec((D, D), lambda i: (0, 0))


_init_tc = pl.pallas_call(
    _init_tc_body,
    grid=(N // _RB,),
    in_specs=[_half_spec(), _row_spec()],
    out_specs=[_row_spec(), _half_spec()],
    out_shape=[
        jax.ShapeDtypeStruct((N, D), jnp.float32),
        jax.ShapeDtypeStruct((NC, ROWS_PAD, D), jnp.float32),
    ],
)

_layer_tc = pl.pallas_call(
    _layer_tc_body,
    grid=(N // _RB,),
    in_specs=[_row_spec(), _half_spec(), _row_spec(), _w_spec(), _w_spec(),
              _row_spec()],
    out_specs=[_row_spec(), _half_spec(), _row_spec()],
    out_shape=[
        jax.ShapeDtypeStruct((N, D), jnp.float32),
        jax.ShapeDtypeStruct((NC, ROWS_PAD, D), jnp.float32),
        jax.ShapeDtypeStruct((N, D), jnp.float32),
    ],
)


def kernel(user_emb, item_emb, edge_index, W1, W2):
    src = edge_index[0]
    dst = edge_index[1]
    npad = E_PAD - E
    pad_r = jnp.full((npad,), PAD_ROW, dtype=jnp.int32)
    pad_c = jnp.zeros((npad,), dtype=jnp.int32)
    rows_all = jnp.stack([
        jnp.concatenate([src, pad_r]),
        jnp.concatenate([dst, pad_r]),
    ]).reshape(NC, NS, NCHUNK, 1, CH)
    cols_all = jnp.stack([
        jnp.concatenate([dst, pad_c]),
        jnp.concatenate([src, pad_c]),
    ]).reshape(NC, NS, NCHUNK, 1, CH)
    edges = jnp.concatenate([cols_all, rows_all], axis=3)

    onesD = jnp.ones((CH, D), jnp.float32)
    zerosD = jnp.zeros((RPT, D), jnp.float32)

    _deg_sc, _spmm_sc = _build_sc_kernels()
    deg_sc = _deg_sc(edges, onesD, zerosD)

    ego = jnp.concatenate([user_emb, item_emb], axis=0)
    dinv, xs = _init_tc(deg_sc, ego)

    acc = ego
    for k in range(L):
        s_raw = _spmm_sc(xs, edges, zerosD)
        ego, xs, acc = _layer_tc(ego, s_raw, dinv, W1[k], W2[k], acc)

    mean = acc * 0.25
    return (mean[:N_U], mean[N_U:])
